# Initial kernel scaffold; baseline (speedup 1.0000x reference)
import jax
import jax.numpy as jnp
from jax.experimental import pallas as pl


def kernel(x, edge_index, batch, W1, b1, W2, b2):
    raise NotImplementedError("write your pallas kernel here")


# speedup vs baseline: 33.7278x; 33.7278x over previous
import functools

import jax
import jax.numpy as jnp
from jax import lax
from jax.experimental import pallas as pl
from jax.experimental.pallas import tpu as pltpu
from jax.experimental.pallas import tpu_sc as plsc

NC = 2
NS = 16
NW = NC * NS
CH = 128


@functools.lru_cache(maxsize=None)
def _make_deg_kernel(nch, npad):
    rpt = npad // NS
    mesh = plsc.VectorSubcoreMesh(core_axis_name="c", subcore_axis_name="s", num_cores=NC, num_subcores=NS)

    @functools.partial(
        pl.kernel,
        out_type=jax.ShapeDtypeStruct((NC, npad), jnp.float32),
        mesh=mesh,
        compiler_params=pltpu.CompilerParams(use_tc_tiling_on_sc=False),
        scratch_types=[
            pltpu.VMEM((nch, CH), jnp.int32),
            pltpu.VMEM((CH,), jnp.float32),
            pltpu.VMEM((rpt,), jnp.float32),
            pltpu.VMEM_SHARED((npad,), jnp.float32),
        ],
    )
    def deg_kernel(dst_hbm, out_hbm, dst_v, ones_v, z_v, acc_sh):
        cid = lax.axis_index("c")
        sid = lax.axis_index("s")
        wid = cid * NS + sid
        pltpu.sync_copy(dst_hbm.at[wid], dst_v)
        for i in range(CH // 16):
            ones_v[pl.ds(i * 16, 16)] = jnp.ones((16,), jnp.float32)

        def zero_body(i, c):
            z_v[pl.ds(i * 16, 16)] = jnp.zeros((16,), jnp.float32)
            return c

        lax.fori_loop(0, rpt // 16, zero_body, 0)
        pltpu.sync_copy(z_v, acc_sh.at[pl.ds(sid * rpt, rpt)])
        plsc.subcore_barrier()

        def body(j, c):
            pltpu.sync_copy(ones_v, acc_sh.at[dst_v.at[j]], add=True)
            return c

        lax.fori_loop(0, nch, body, 0)
        plsc.subcore_barrier()
        pltpu.sync_copy(acc_sh.at[pl.ds(sid * rpt, rpt)],
                        out_hbm.at[cid, pl.ds(sid * rpt, rpt)])

    return deg_kernel


@functools.lru_cache(maxsize=None)
def _make_edge_kernel(nch, npad, feat):
    rpt = npad // NS
    mesh = plsc.VectorSubcoreMesh(core_axis_name="c", subcore_axis_name="s", num_cores=NC, num_subcores=NS)

    @functools.partial(
        pl.kernel,
        out_type=jax.ShapeDtypeStruct((NC, npad, feat), jnp.float32),
        mesh=mesh,
        compiler_params=pltpu.CompilerParams(use_tc_tiling_on_sc=False),
        scratch_types=[
            pltpu.VMEM((nch, CH), jnp.int32),
            pltpu.VMEM((nch, CH), jnp.int32),
            pltpu.VMEM((CH, feat), jnp.float32),
            pltpu.VMEM((rpt, feat), jnp.float32),
            pltpu.VMEM_SHARED((npad, feat), jnp.float32),
        ],
    )
    def edge_kernel(src_hbm, dst_hbm, g_hbm, out_hbm,
                    src_v, dst_v, rows_v, z_v, acc_sh):
        cid = lax.axis_index("c")
        sid = lax.axis_index("s")
        wid = cid * NS + sid
        pltpu.sync_copy(src_hbm.at[wid], src_v)
        pltpu.sync_copy(dst_hbm.at[wid], dst_v)

        def zero_body(i, c):
            z_v[i] = jnp.zeros((feat,), jnp.float32)
            return c

        lax.fori_loop(0, rpt, zero_body, 0)
        pltpu.sync_copy(z_v, acc_sh.at[pl.ds(sid * rpt, rpt)])
        plsc.subcore_barrier()

        def body(j, c):
            pltpu.sync_copy(g_hbm.at[src_v.at[j]], rows_v)
            pltpu.sync_copy(rows_v, acc_sh.at[dst_v.at[j]], add=True)
            return c

        lax.fori_loop(0, nch, body, 0)
        plsc.subcore_barrier()
        pltpu.sync_copy(acc_sh.at[pl.ds(sid * rpt, rpt)],
                        out_hbm.at[cid, pl.ds(sid * rpt, rpt)])

    return edge_kernel


def _tc1_body(dp_ref, x_ref, w_ref, g_ref):
    deg = dp_ref[0] + dp_ref[1] + 1.0
    dinv = lax.rsqrt(deg)
    h = jnp.dot(x_ref[...], w_ref[...], preferred_element_type=jnp.float32)
    g_ref[...] = h * dinv


def _tc2_body(sp_ref, dp_ref, g1_ref, b1_ref, g2_ref):
    deg = dp_ref[0] + dp_ref[1] + 1.0
    dinv = lax.rsqrt(deg)
    s = sp_ref[0] + sp_ref[1] + g1_ref[...]
    h1 = jnp.maximum(s * dinv + b1_ref[...], 0.0)
    g2_ref[...] = h1 * dinv


def _tc3_body(sp_ref, dp_ref, g2_ref, w2_ref, b2_ref, o_ref):
    deg = dp_ref[0] + dp_ref[1] + 1.0
    dinv = lax.rsqrt(deg)
    s = sp_ref[0] + sp_ref[1] + g2_ref[...]
    t = jnp.dot(s, w2_ref[...], preferred_element_type=jnp.float32)
    o_ref[...] = jax.nn.sigmoid(t * dinv + b2_ref[...])


def kernel(x, edge_index, batch, W1, b1, W2, b2):
    n, f = x.shape
    h = W1.shape[1]
    e = edge_index.shape[1]
    npad = -(-(n + 1) // 256) * 256
    ew = -(-e // NW)
    nch = -(-ew // CH)
    tot = NW * nch * CH

    src = edge_index[0]
    dst = edge_index[1]
    src3 = jnp.concatenate(
        [src, jnp.zeros((tot - e,), jnp.int32)]).reshape(NW, nch, CH)
    dst3 = jnp.concatenate(
        [dst, jnp.full((tot - e,), n, jnp.int32)]).reshape(NW, nch, CH)

    deg_k = _make_deg_kernel(nch, npad)
    edge_k = _make_edge_kernel(nch, npad, h)

    degp = deg_k(dst3)
    dp = degp.reshape(NC, npad, 1)

    blk = 1000
    grid = (n // blk,)
    dp_spec = pl.BlockSpec((NC, blk, 1), lambda i: (0, i, 0))
    sp_spec = pl.BlockSpec((NC, blk, h), lambda i: (0, i, 0))
    g_spec = pl.BlockSpec((blk, h), lambda i: (i, 0))

    g1 = pl.pallas_call(
        _tc1_body,
        grid=grid,
        in_specs=[dp_spec,
                  pl.BlockSpec((blk, f), lambda i: (i, 0)),
                  pl.BlockSpec((f, h), lambda i: (0, 0))],
        out_specs=g_spec,
        out_shape=jax.ShapeDtypeStruct((n, h), jnp.float32),
    )(dp, x, W1)

    s1p = edge_k(src3, dst3, g1)

    g2 = pl.pallas_call(
        _tc2_body,
        grid=grid,
        in_specs=[sp_spec, dp_spec, g_spec,
                  pl.BlockSpec((1, h), lambda i: (0, 0))],
        out_specs=g_spec,
        out_shape=jax.ShapeDtypeStruct((n, h), jnp.float32),
    )(s1p, dp, g1, b1.reshape(1, h))

    s2p = edge_k(src3, dst3, g2)

    out = pl.pallas_call(
        _tc3_body,
        grid=grid,
        in_specs=[sp_spec, dp_spec, g_spec,
                  pl.BlockSpec((h, 1), lambda i: (0, 0)),
                  pl.BlockSpec((1, 1), lambda i: (0, 0))],
        out_specs=pl.BlockSpec((blk, 1), lambda i: (i, 0)),
        out_shape=jax.ShapeDtypeStruct((n, 1), jnp.float32),
    )(s2p, dp, g2, W2, b2.reshape(1, 1))

    return out


# speedup vs baseline: 41.8521x; 1.2409x over previous
import functools

import jax
import jax.numpy as jnp
from jax import lax
from jax.experimental import pallas as pl
from jax.experimental.pallas import tpu as pltpu
from jax.experimental.pallas import tpu_sc as plsc

NC = 2
NS = 16
NW = NC * NS
CH = 128


@functools.lru_cache(maxsize=None)
def _make_deg_kernel(nch, npad):
    rpt = npad // NS
    mesh = plsc.VectorSubcoreMesh(core_axis_name="c", subcore_axis_name="s", num_cores=NC, num_subcores=NS)

    @functools.partial(
        pl.kernel,
        out_type=jax.ShapeDtypeStruct((NC, npad), jnp.float32),
        mesh=mesh,
        compiler_params=pltpu.CompilerParams(use_tc_tiling_on_sc=False),
        scratch_types=[
            pltpu.VMEM((nch, CH), jnp.int32),
            pltpu.VMEM((CH,), jnp.float32),
            pltpu.VMEM((rpt,), jnp.float32),
            pltpu.VMEM_SHARED((npad,), jnp.float32),
        ],
    )
    def deg_kernel(dst_hbm, out_hbm, dst_v, ones_v, z_v, acc_sh):
        cid = lax.axis_index("c")
        sid = lax.axis_index("s")
        wid = cid * NS + sid
        pltpu.sync_copy(dst_hbm.at[wid], dst_v)
        for i in range(CH // 16):
            ones_v[pl.ds(i * 16, 16)] = jnp.ones((16,), jnp.float32)

        def zero_body(i, c):
            z_v[pl.ds(i * 16, 16)] = jnp.zeros((16,), jnp.float32)
            return c

        lax.fori_loop(0, rpt // 16, zero_body, 0)
        pltpu.sync_copy(z_v, acc_sh.at[pl.ds(sid * rpt, rpt)])
        plsc.subcore_barrier()

        def body(j, c):
            pltpu.sync_copy(ones_v, acc_sh.at[dst_v.at[j]], add=True)
            return c

        lax.fori_loop(0, nch, body, 0)
        plsc.subcore_barrier()
        pltpu.sync_copy(acc_sh.at[pl.ds(sid * rpt, rpt)],
                        out_hbm.at[cid, pl.ds(sid * rpt, rpt)])

    return deg_kernel


@functools.lru_cache(maxsize=None)
def _make_edge_kernel(nch, npad, feat):
    rpt = npad // NS
    mesh = plsc.VectorSubcoreMesh(core_axis_name="c", subcore_axis_name="s", num_cores=NC, num_subcores=NS)

    @functools.partial(
        pl.kernel,
        out_type=jax.ShapeDtypeStruct((NC, npad, feat), jnp.float32),
        mesh=mesh,
        compiler_params=pltpu.CompilerParams(use_tc_tiling_on_sc=False),
        scratch_types=[
            pltpu.VMEM((nch, CH), jnp.int32),
            pltpu.VMEM((nch, CH), jnp.int32),
            pltpu.VMEM((CH, feat), jnp.float32),
            pltpu.VMEM((CH, feat), jnp.float32),
            pltpu.VMEM((rpt, feat), jnp.float32),
            pltpu.VMEM_SHARED((npad, feat), jnp.float32),
            pltpu.SemaphoreType.DMA,
            pltpu.SemaphoreType.DMA,
        ],
    )
    def edge_kernel(src_hbm, dst_hbm, g_hbm, out_hbm,
                    src_v, dst_v, rows0_v, rows1_v, z_v, acc_sh,
                    sem0, sem1):
        cid = lax.axis_index("c")
        sid = lax.axis_index("s")
        wid = cid * NS + sid
        pltpu.sync_copy(src_hbm.at[wid], src_v)
        pltpu.sync_copy(dst_hbm.at[wid], dst_v)

        def zero_body(i, c):
            z_v[i] = jnp.zeros((feat,), jnp.float32)
            return c

        lax.fori_loop(0, rpt, zero_body, 0)
        pltpu.sync_copy(z_v, acc_sh.at[pl.ds(sid * rpt, rpt)])
        plsc.subcore_barrier()

        last = nch - 2
        pltpu.async_copy(g_hbm.at[src_v.at[0]], rows0_v, sem0)
        pltpu.async_copy(g_hbm.at[src_v.at[1]], rows1_v, sem1)

        def body(i, c):
            j0 = 2 * i
            j1 = j0 + 1
            n0 = jnp.minimum(j0 + 2, last)
            n1 = jnp.minimum(j1 + 2, last + 1)
            pltpu.make_async_copy(g_hbm.at[src_v.at[j0]], rows0_v, sem0).wait()
            pltpu.sync_copy(rows0_v, acc_sh.at[dst_v.at[j0]], add=True)
            pltpu.async_copy(g_hbm.at[src_v.at[n0]], rows0_v, sem0)
            pltpu.make_async_copy(g_hbm.at[src_v.at[j1]], rows1_v, sem1).wait()
            pltpu.sync_copy(rows1_v, acc_sh.at[dst_v.at[j1]], add=True)
            pltpu.async_copy(g_hbm.at[src_v.at[n1]], rows1_v, sem1)
            return c

        lax.fori_loop(0, nch // 2, body, 0)
        pltpu.make_async_copy(g_hbm.at[src_v.at[last]], rows0_v, sem0).wait()
        pltpu.make_async_copy(g_hbm.at[src_v.at[last + 1]], rows1_v, sem1).wait()
        plsc.subcore_barrier()
        pltpu.sync_copy(acc_sh.at[pl.ds(sid * rpt, rpt)],
                        out_hbm.at[cid, pl.ds(sid * rpt, rpt)])

    return edge_kernel


def _tc1_body(dp_ref, x_ref, w_ref, g_ref):
    deg = dp_ref[0] + dp_ref[1] + 1.0
    dinv = lax.rsqrt(deg)
    h = jnp.dot(x_ref[...], w_ref[...], preferred_element_type=jnp.float32)
    g_ref[...] = h * dinv


def _tc2_body(sp_ref, dp_ref, g1_ref, b1_ref, g2_ref):
    deg = dp_ref[0] + dp_ref[1] + 1.0
    dinv = lax.rsqrt(deg)
    s = sp_ref[0] + sp_ref[1] + g1_ref[...]
    h1 = jnp.maximum(s * dinv + b1_ref[...], 0.0)
    g2_ref[...] = h1 * dinv


def _tc3_body(sp_ref, dp_ref, g2_ref, w2_ref, b2_ref, o_ref):
    deg = dp_ref[0] + dp_ref[1] + 1.0
    dinv = lax.rsqrt(deg)
    s = sp_ref[0] + sp_ref[1] + g2_ref[...]
    t = jnp.dot(s, w2_ref[...], preferred_element_type=jnp.float32)
    o_ref[...] = jax.nn.sigmoid(t * dinv + b2_ref[...])


def kernel(x, edge_index, batch, W1, b1, W2, b2):
    n, f = x.shape
    h = W1.shape[1]
    e = edge_index.shape[1]
    npad = -(-(n + 1) // 256) * 256
    ew = -(-e // NW)
    nch = -(-ew // CH)
    nch += nch % 2
    tot = NW * nch * CH

    src = edge_index[0]
    dst = edge_index[1]
    junk = n + (jnp.arange(tot - e, dtype=jnp.int32) % (npad - n))
    src3 = jnp.concatenate(
        [src, jnp.zeros((tot - e,), jnp.int32)]).reshape(NW, nch, CH)
    dst3 = jnp.concatenate([dst, junk]).reshape(NW, nch, CH)

    deg_k = _make_deg_kernel(nch, npad)
    edge_k = _make_edge_kernel(nch, npad, h)

    degp = deg_k(dst3)
    dp = degp.reshape(NC, npad, 1)

    blk = 1000
    grid = (n // blk,)
    dp_spec = pl.BlockSpec((NC, blk, 1), lambda i: (0, i, 0))
    sp_spec = pl.BlockSpec((NC, blk, h), lambda i: (0, i, 0))
    g_spec = pl.BlockSpec((blk, h), lambda i: (i, 0))

    g1 = pl.pallas_call(
        _tc1_body,
        grid=grid,
        in_specs=[dp_spec,
                  pl.BlockSpec((blk, f), lambda i: (i, 0)),
                  pl.BlockSpec((f, h), lambda i: (0, 0))],
        out_specs=g_spec,
        out_shape=jax.ShapeDtypeStruct((n, h), jnp.float32),
    )(dp, x, W1)

    s1p = edge_k(src3, dst3, g1)

    g2 = pl.pallas_call(
        _tc2_body,
        grid=grid,
        in_specs=[sp_spec, dp_spec, g_spec,
                  pl.BlockSpec((1, h), lambda i: (0, 0))],
        out_specs=g_spec,
        out_shape=jax.ShapeDtypeStruct((n, h), jnp.float32),
    )(s1p, dp, g1, b1.reshape(1, h))

    s2p = edge_k(src3, dst3, g2)

    out = pl.pallas_call(
        _tc3_body,
        grid=grid,
        in_specs=[sp_spec, dp_spec, g_spec,
                  pl.BlockSpec((h, 1), lambda i: (0, 0)),
                  pl.BlockSpec((1, 1), lambda i: (0, 0))],
        out_specs=pl.BlockSpec((blk, 1), lambda i: (i, 0)),
        out_shape=jax.ShapeDtypeStruct((n, 1), jnp.float32),
    )(s2p, dp, g2, W2, b2.reshape(1, 1))

    return out


# speedup vs baseline: 42.3922x; 1.0129x over previous
import functools

import jax
import jax.numpy as jnp
from jax import lax
from jax.experimental import pallas as pl
from jax.experimental.pallas import tpu as pltpu
from jax.experimental.pallas import tpu_sc as plsc

NC = 2
NS = 16
NW = NC * NS
CH = 128


@functools.lru_cache(maxsize=None)
def _make_deg_kernel(nch, npad):
    rpt = npad // NS
    mesh = plsc.VectorSubcoreMesh(core_axis_name="c", subcore_axis_name="s", num_cores=NC, num_subcores=NS)

    @functools.partial(
        pl.kernel,
        out_type=jax.ShapeDtypeStruct((NC, npad), jnp.float32),
        mesh=mesh,
        compiler_params=pltpu.CompilerParams(use_tc_tiling_on_sc=False),
        scratch_types=[
            pltpu.VMEM((nch, CH), jnp.int32),
            pltpu.VMEM((CH,), jnp.float32),
            pltpu.VMEM((rpt,), jnp.float32),
            pltpu.VMEM_SHARED((npad,), jnp.float32),
        ],
    )
    def deg_kernel(dst_hbm, out_hbm, dst_v, ones_v, z_v, acc_sh):
        cid = lax.axis_index("c")
        sid = lax.axis_index("s")
        wid = cid * NS + sid
        pltpu.sync_copy(dst_hbm.at[wid], dst_v)
        for i in range(CH // 16):
            ones_v[pl.ds(i * 16, 16)] = jnp.ones((16,), jnp.float32)

        def zero_body(i, c):
            z_v[pl.ds(i * 16, 16)] = jnp.zeros((16,), jnp.float32)
            return c

        lax.fori_loop(0, rpt // 16, zero_body, 0)
        pltpu.sync_copy(z_v, acc_sh.at[pl.ds(sid * rpt, rpt)])
        plsc.subcore_barrier()

        def body(j, c):
            pltpu.sync_copy(ones_v, acc_sh.at[dst_v.at[j]], add=True)
            return c

        lax.fori_loop(0, nch, body, 0)
        plsc.subcore_barrier()
        pltpu.sync_copy(acc_sh.at[pl.ds(sid * rpt, rpt)],
                        out_hbm.at[cid, pl.ds(sid * rpt, rpt)])

    return deg_kernel


@functools.lru_cache(maxsize=None)
def _make_edge_kernel(nch, npad, feat):
    rpt = npad // NS
    mesh = plsc.VectorSubcoreMesh(core_axis_name="c", subcore_axis_name="s", num_cores=NC, num_subcores=NS)

    @functools.partial(
        pl.kernel,
        out_type=jax.ShapeDtypeStruct((NC, npad, feat), jnp.float32),
        mesh=mesh,
        compiler_params=pltpu.CompilerParams(use_tc_tiling_on_sc=False),
        scratch_types=[
            pltpu.VMEM((nch, CH), jnp.int32),
            pltpu.VMEM((nch, CH), jnp.int32),
            pltpu.VMEM((8, CH, feat), jnp.float32),
            pltpu.VMEM((rpt, feat), jnp.float32),
            pltpu.VMEM_SHARED((npad, feat), jnp.float32),
            pltpu.SemaphoreType.DMA,
            pltpu.SemaphoreType.DMA,
            pltpu.SemaphoreType.DMA,
            pltpu.SemaphoreType.DMA,
        ],
    )
    def edge_kernel(src_hbm, dst_hbm, g_hbm, out_hbm,
                    src_v, dst_v, rows_v, z_v, acc_sh,
                    gsem_a, gsem_b, ssem_a, ssem_b):
        cid = lax.axis_index("c")
        sid = lax.axis_index("s")
        wid = cid * NS + sid
        pltpu.sync_copy(src_hbm.at[wid], src_v)
        pltpu.sync_copy(dst_hbm.at[wid], dst_v)

        def zero_body(i, c):
            z_v[i] = jnp.zeros((feat,), jnp.float32)
            return c

        lax.fori_loop(0, rpt, zero_body, 0)
        pltpu.sync_copy(z_v, acc_sh.at[pl.ds(sid * rpt, rpt)])
        plsc.subcore_barrier()

        sems = {"A": (gsem_a, ssem_a), "B": (gsem_b, ssem_b)}

        def issue_group(grp, base):
            gsem, _ = sems[grp]
            off = 0 if grp == "A" else 4
            for b in range(4):
                pltpu.async_copy(g_hbm.at[src_v.at[base + b]],
                                 rows_v.at[off + b], gsem)

        def process_group(grp, j, nxt):
            gsem, ssem = sems[grp]
            off = 0 if grp == "A" else 4
            for b in range(4):
                pltpu.make_async_copy(g_hbm.at[src_v.at[j + b]],
                                      rows_v.at[off + b], gsem).wait()
            for b in range(4):
                pltpu.async_copy(rows_v.at[off + b],
                                 acc_sh.at[dst_v.at[j + b]], ssem, add=True)
            for b in range(4):
                pltpu.make_async_copy(rows_v.at[off + b],
                                      acc_sh.at[dst_v.at[j + b]], ssem).wait()
            for b in range(4):
                pltpu.async_copy(g_hbm.at[src_v.at[jnp.minimum(nxt + b,
                                                               nch - 1)]],
                                 rows_v.at[off + b], gsem)

        issue_group("A", 0)
        issue_group("B", 4)

        def body(i, c):
            j = 8 * i
            process_group("A", j, j + 8)
            process_group("B", j + 4, j + 12)
            return c

        lax.fori_loop(0, nch // 8, body, 0)
        for b in range(4):
            pltpu.make_async_copy(g_hbm.at[src_v.at[nch - 1]],
                                  rows_v.at[b], gsem_a).wait()
        for b in range(4):
            pltpu.make_async_copy(g_hbm.at[src_v.at[nch - 1]],
                                  rows_v.at[4 + b], gsem_b).wait()
        plsc.subcore_barrier()
        pltpu.sync_copy(acc_sh.at[pl.ds(sid * rpt, rpt)],
                        out_hbm.at[cid, pl.ds(sid * rpt, rpt)])

    return edge_kernel


def _tc1_body(dp_ref, x_ref, w_ref, g_ref):
    deg = dp_ref[0] + dp_ref[1] + 1.0
    dinv = lax.rsqrt(deg)
    h = jnp.dot(x_ref[...], w_ref[...], preferred_element_type=jnp.float32)
    g_ref[...] = h * dinv


def _tc2_body(sp_ref, dp_ref, g1_ref, b1_ref, g2_ref):
    deg = dp_ref[0] + dp_ref[1] + 1.0
    dinv = lax.rsqrt(deg)
    s = sp_ref[0] + sp_ref[1] + g1_ref[...]
    h1 = jnp.maximum(s * dinv + b1_ref[...], 0.0)
    g2_ref[...] = h1 * dinv


def _tc3_body(sp_ref, dp_ref, g2_ref, w2_ref, b2_ref, o_ref):
    deg = dp_ref[0] + dp_ref[1] + 1.0
    dinv = lax.rsqrt(deg)
    s = sp_ref[0] + sp_ref[1] + g2_ref[...]
    t = jnp.dot(s, w2_ref[...], preferred_element_type=jnp.float32)
    o_ref[...] = jax.nn.sigmoid(t * dinv + b2_ref[...])


def kernel(x, edge_index, batch, W1, b1, W2, b2):
    n, f = x.shape
    h = W1.shape[1]
    e = edge_index.shape[1]
    npad = -(-(n + 1) // 256) * 256
    ew = -(-e // NW)
    nch = -(-ew // CH)
    nch = -(-nch // 8) * 8
    tot = NW * nch * CH

    src = edge_index[0]
    dst = edge_index[1]
    junk = n + (jnp.arange(tot - e, dtype=jnp.int32) % (npad - n))
    src3 = jnp.concatenate(
        [src, jnp.zeros((tot - e,), jnp.int32)]).reshape(NW, nch, CH)
    dst3 = jnp.concatenate([dst, junk]).reshape(NW, nch, CH)

    deg_k = _make_deg_kernel(nch, npad)
    edge_k = _make_edge_kernel(nch, npad, h)

    degp = deg_k(dst3)
    dp = degp.reshape(NC, npad, 1)

    blk = 1000
    grid = (n // blk,)
    dp_spec = pl.BlockSpec((NC, blk, 1), lambda i: (0, i, 0))
    sp_spec = pl.BlockSpec((NC, blk, h), lambda i: (0, i, 0))
    g_spec = pl.BlockSpec((blk, h), lambda i: (i, 0))

    g1 = pl.pallas_call(
        _tc1_body,
        grid=grid,
        in_specs=[dp_spec,
                  pl.BlockSpec((blk, f), lambda i: (i, 0)),
                  pl.BlockSpec((f, h), lambda i: (0, 0))],
        out_specs=g_spec,
        out_shape=jax.ShapeDtypeStruct((n, h), jnp.float32),
    )(dp, x, W1)

    s1p = edge_k(src3, dst3, g1)

    g2 = pl.pallas_call(
        _tc2_body,
        grid=grid,
        in_specs=[sp_spec, dp_spec, g_spec,
                  pl.BlockSpec((1, h), lambda i: (0, 0))],
        out_specs=g_spec,
        out_shape=jax.ShapeDtypeStruct((n, h), jnp.float32),
    )(s1p, dp, g1, b1.reshape(1, h))

    s2p = edge_k(src3, dst3, g2)

    out = pl.pallas_call(
        _tc3_body,
        grid=grid,
        in_specs=[sp_spec, dp_spec, g_spec,
                  pl.BlockSpec((h, 1), lambda i: (0, 0)),
                  pl.BlockSpec((1, 1), lambda i: (0, 0))],
        out_specs=pl.BlockSpec((blk, 1), lambda i: (i, 0)),
        out_shape=jax.ShapeDtypeStruct((n, 1), jnp.float32),
    )(s2p, dp, g2, W2, b2.reshape(1, 1))

    return out


# speedup vs baseline: 57.6053x; 1.3589x over previous
import functools

import jax
import jax.numpy as jnp
from jax import lax
from jax.experimental import pallas as pl
from jax.experimental.pallas import tpu as pltpu
from jax.experimental.pallas import tpu_sc as plsc

NC = 2
NS = 16
NW = NC * NS
CH = 128


@functools.lru_cache(maxsize=None)
def _make_deg_kernel(nch, npad):
    rpt = npad // NS
    mesh = plsc.VectorSubcoreMesh(core_axis_name="c", subcore_axis_name="s", num_cores=NC, num_subcores=NS)

    @functools.partial(
        pl.kernel,
        out_type=jax.ShapeDtypeStruct((NC, npad), jnp.float32),
        mesh=mesh,
        compiler_params=pltpu.CompilerParams(use_tc_tiling_on_sc=False),
        scratch_types=[
            pltpu.VMEM((nch, CH), jnp.int32),
            pltpu.VMEM((CH,), jnp.float32),
            pltpu.VMEM((rpt,), jnp.float32),
            pltpu.VMEM_SHARED((npad,), jnp.float32),
        ],
    )
    def deg_kernel(dst_hbm, out_hbm, dst_v, ones_v, z_v, acc_sh):
        cid = lax.axis_index("c")
        sid = lax.axis_index("s")
        wid = cid * NS + sid
        pltpu.sync_copy(dst_hbm.at[wid], dst_v)
        for i in range(CH // 16):
            ones_v[pl.ds(i * 16, 16)] = jnp.ones((16,), jnp.float32)

        def zero_body(i, c):
            z_v[pl.ds(i * 16, 16)] = jnp.zeros((16,), jnp.float32)
            return c

        lax.fori_loop(0, rpt // 16, zero_body, 0)
        pltpu.sync_copy(z_v, acc_sh.at[pl.ds(sid * rpt, rpt)])
        plsc.subcore_barrier()

        def body(j, c):
            pltpu.sync_copy(ones_v, acc_sh.at[dst_v.at[j]], add=True)
            return c

        lax.fori_loop(0, nch, body, 0)
        plsc.subcore_barrier()
        pltpu.sync_copy(acc_sh.at[pl.ds(sid * rpt, rpt)],
                        out_hbm.at[cid, pl.ds(sid * rpt, rpt)])

    return deg_kernel


@functools.lru_cache(maxsize=None)
def _make_edge_kernel(nch, npad, feat, n_g):
    rpt = npad // NS
    rpt_g = n_g // NS
    mesh = plsc.VectorSubcoreMesh(core_axis_name="c", subcore_axis_name="s", num_cores=NC, num_subcores=NS)

    @functools.partial(
        pl.kernel,
        out_type=jax.ShapeDtypeStruct((NC, npad, feat), jnp.float32),
        mesh=mesh,
        compiler_params=pltpu.CompilerParams(use_tc_tiling_on_sc=False),
        scratch_types=[
            pltpu.VMEM((nch, CH), jnp.int32),
            pltpu.VMEM((nch, CH), jnp.int32),
            pltpu.VMEM((8, CH, feat), jnp.float32),
            pltpu.VMEM((rpt, feat), jnp.float32),
            pltpu.VMEM_SHARED((npad, feat), jnp.float32),
            pltpu.VMEM_SHARED((n_g, feat), jnp.float32),
            pltpu.SemaphoreType.DMA,
            pltpu.SemaphoreType.DMA,
            pltpu.SemaphoreType.DMA,
            pltpu.SemaphoreType.DMA,
        ],
    )
    def edge_kernel(src_hbm, dst_hbm, g_hbm, out_hbm,
                    src_v, dst_v, rows_v, z_v, acc_sh, g_sh,
                    gsem_a, gsem_b, ssem_a, ssem_b):
        cid = lax.axis_index("c")
        sid = lax.axis_index("s")
        wid = cid * NS + sid
        pltpu.sync_copy(src_hbm.at[wid], src_v)
        pltpu.sync_copy(dst_hbm.at[wid], dst_v)
        pltpu.sync_copy(g_hbm.at[pl.ds(sid * rpt_g, rpt_g)],
                        g_sh.at[pl.ds(sid * rpt_g, rpt_g)])

        def zero_body(i, c):
            z_v[i] = jnp.zeros((feat,), jnp.float32)
            return c

        lax.fori_loop(0, rpt, zero_body, 0)
        pltpu.sync_copy(z_v, acc_sh.at[pl.ds(sid * rpt, rpt)])
        plsc.subcore_barrier()

        sems = {"A": (gsem_a, ssem_a), "B": (gsem_b, ssem_b)}

        def issue_group(grp, base):
            gsem, _ = sems[grp]
            off = 0 if grp == "A" else 4
            for b in range(4):
                pltpu.async_copy(g_sh.at[src_v.at[base + b]],
                                 rows_v.at[off + b], gsem)

        def process_group(grp, j, nxt):
            gsem, ssem = sems[grp]
            off = 0 if grp == "A" else 4
            for b in range(4):
                pltpu.make_async_copy(g_sh.at[src_v.at[j + b]],
                                      rows_v.at[off + b], gsem).wait()
            for b in range(4):
                pltpu.async_copy(rows_v.at[off + b],
                                 acc_sh.at[dst_v.at[j + b]], ssem, add=True)
            for b in range(4):
                pltpu.make_async_copy(rows_v.at[off + b],
                                      acc_sh.at[dst_v.at[j + b]], ssem).wait()
            for b in range(4):
                pltpu.async_copy(g_sh.at[src_v.at[jnp.minimum(nxt + b,
                                                              nch - 1)]],
                                 rows_v.at[off + b], gsem)

        issue_group("A", 0)
        issue_group("B", 4)

        def body(i, c):
            j = 8 * i
            process_group("A", j, j + 8)
            process_group("B", j + 4, j + 12)
            return c

        lax.fori_loop(0, nch // 8, body, 0)
        for b in range(4):
            pltpu.make_async_copy(g_sh.at[src_v.at[nch - 1]],
                                  rows_v.at[b], gsem_a).wait()
        for b in range(4):
            pltpu.make_async_copy(g_sh.at[src_v.at[nch - 1]],
                                  rows_v.at[4 + b], gsem_b).wait()
        plsc.subcore_barrier()
        pltpu.sync_copy(acc_sh.at[pl.ds(sid * rpt, rpt)],
                        out_hbm.at[cid, pl.ds(sid * rpt, rpt)])

    return edge_kernel


def _tc1_body(dp_ref, x_ref, w_ref, g_ref):
    deg = dp_ref[0] + dp_ref[1] + 1.0
    dinv = lax.rsqrt(deg)
    h = jnp.dot(x_ref[...], w_ref[...], preferred_element_type=jnp.float32)
    g_ref[...] = h * dinv


def _tc2_body(sp_ref, dp_ref, g1_ref, b1_ref, g2_ref):
    deg = dp_ref[0] + dp_ref[1] + 1.0
    dinv = lax.rsqrt(deg)
    s = sp_ref[0] + sp_ref[1] + g1_ref[...]
    h1 = jnp.maximum(s * dinv + b1_ref[...], 0.0)
    g2_ref[...] = h1 * dinv


def _tc3_body(sp_ref, dp_ref, g2_ref, w2_ref, b2_ref, o_ref):
    deg = dp_ref[0] + dp_ref[1] + 1.0
    dinv = lax.rsqrt(deg)
    s = sp_ref[0] + sp_ref[1] + g2_ref[...]
    t = jnp.dot(s, w2_ref[...], preferred_element_type=jnp.float32)
    o_ref[...] = jax.nn.sigmoid(t * dinv + b2_ref[...])


def kernel(x, edge_index, batch, W1, b1, W2, b2):
    n, f = x.shape
    h = W1.shape[1]
    e = edge_index.shape[1]
    npad = -(-(n + 1) // 256) * 256
    ew = -(-e // NW)
    nch = -(-ew // CH)
    nch = -(-nch // 8) * 8
    tot = NW * nch * CH

    src = edge_index[0]
    dst = edge_index[1]
    junk = n + (jnp.arange(tot - e, dtype=jnp.int32) % (npad - n))
    src3 = jnp.concatenate(
        [src, jnp.zeros((tot - e,), jnp.int32)]).reshape(NW, nch, CH)
    dst3 = jnp.concatenate([dst, junk]).reshape(NW, nch, CH)

    deg_k = _make_deg_kernel(nch, npad)
    edge_k = _make_edge_kernel(nch, npad, h, n)

    degp = deg_k(dst3)
    dp = degp.reshape(NC, npad, 1)

    blk = 1000
    grid = (n // blk,)
    dp_spec = pl.BlockSpec((NC, blk, 1), lambda i: (0, i, 0))
    sp_spec = pl.BlockSpec((NC, blk, h), lambda i: (0, i, 0))
    g_spec = pl.BlockSpec((blk, h), lambda i: (i, 0))

    g1 = pl.pallas_call(
        _tc1_body,
        grid=grid,
        in_specs=[dp_spec,
                  pl.BlockSpec((blk, f), lambda i: (i, 0)),
                  pl.BlockSpec((f, h), lambda i: (0, 0))],
        out_specs=g_spec,
        out_shape=jax.ShapeDtypeStruct((n, h), jnp.float32),
    )(dp, x, W1)

    s1p = edge_k(src3, dst3, g1)

    g2 = pl.pallas_call(
        _tc2_body,
        grid=grid,
        in_specs=[sp_spec, dp_spec, g_spec,
                  pl.BlockSpec((1, h), lambda i: (0, 0))],
        out_specs=g_spec,
        out_shape=jax.ShapeDtypeStruct((n, h), jnp.float32),
    )(s1p, dp, g1, b1.reshape(1, h))

    s2p = edge_k(src3, dst3, g2)

    out = pl.pallas_call(
        _tc3_body,
        grid=grid,
        in_specs=[sp_spec, dp_spec, g_spec,
                  pl.BlockSpec((h, 1), lambda i: (0, 0)),
                  pl.BlockSpec((1, 1), lambda i: (0, 0))],
        out_specs=pl.BlockSpec((blk, 1), lambda i: (i, 0)),
        out_shape=jax.ShapeDtypeStruct((n, 1), jnp.float32),
    )(s2p, dp, g2, W2, b2.reshape(1, 1))

    return out


# speedup vs baseline: 59.9471x; 1.0407x over previous
import functools

import jax
import jax.numpy as jnp
from jax import lax
from jax.experimental import pallas as pl
from jax.experimental.pallas import tpu as pltpu
from jax.experimental.pallas import tpu_sc as plsc

NC = 2
NS = 16
NW = NC * NS
CH = 128


@functools.lru_cache(maxsize=None)
def _make_deg_kernel(nch, npad):
    rpt = npad // NS
    mesh = plsc.VectorSubcoreMesh(core_axis_name="c", subcore_axis_name="s", num_cores=NC, num_subcores=NS)

    @functools.partial(
        pl.kernel,
        out_type=jax.ShapeDtypeStruct((NC, npad), jnp.float32),
        mesh=mesh,
        compiler_params=pltpu.CompilerParams(use_tc_tiling_on_sc=False),
        scratch_types=[
            pltpu.VMEM((nch, CH), jnp.int32),
            pltpu.VMEM((CH,), jnp.float32),
            pltpu.VMEM((rpt,), jnp.float32),
            pltpu.VMEM_SHARED((npad,), jnp.float32),
        ],
    )
    def deg_kernel(dst_hbm, out_hbm, dst_v, ones_v, z_v, acc_sh):
        cid = lax.axis_index("c")
        sid = lax.axis_index("s")
        wid = cid * NS + sid
        pltpu.sync_copy(dst_hbm.at[wid], dst_v)
        for i in range(CH // 16):
            ones_v[pl.ds(i * 16, 16)] = jnp.ones((16,), jnp.float32)

        def zero_body(i, c):
            z_v[pl.ds(i * 16, 16)] = jnp.zeros((16,), jnp.float32)
            return c

        lax.fori_loop(0, rpt // 16, zero_body, 0)
        pltpu.sync_copy(z_v, acc_sh.at[pl.ds(sid * rpt, rpt)])
        plsc.subcore_barrier()

        def body(j, c):
            pltpu.sync_copy(ones_v, acc_sh.at[dst_v.at[j]], add=True)
            return c

        lax.fori_loop(0, nch, body, 0)
        plsc.subcore_barrier()
        pltpu.sync_copy(acc_sh.at[pl.ds(sid * rpt, rpt)],
                        out_hbm.at[cid, pl.ds(sid * rpt, rpt)])

    return deg_kernel


@functools.lru_cache(maxsize=None)
def _make_edge_kernel(nch, npad, feat, n_g):
    rpt = npad // NS
    rpt_g = -(-n_g // NS) // 8 * 8 + 8
    mesh = plsc.VectorSubcoreMesh(core_axis_name="c", subcore_axis_name="s", num_cores=NC, num_subcores=NS)

    @functools.partial(
        pl.kernel,
        out_type=jax.ShapeDtypeStruct((NC, npad, feat), jnp.float32),
        mesh=mesh,
        compiler_params=pltpu.CompilerParams(use_tc_tiling_on_sc=False),
        scratch_types=[
            pltpu.VMEM((nch, CH), jnp.int32),
            pltpu.VMEM((nch, CH), jnp.int32),
            pltpu.VMEM((8, CH, feat), jnp.float32),
            pltpu.VMEM((rpt, feat), jnp.float32),
            pltpu.VMEM_SHARED((npad, feat), jnp.float32),
            pltpu.VMEM_SHARED((n_g, feat), jnp.float32),
            pltpu.SemaphoreType.DMA,
            pltpu.SemaphoreType.DMA,
            pltpu.SemaphoreType.DMA,
            pltpu.SemaphoreType.DMA,
        ],
    )
    def edge_kernel(src_hbm, dst_hbm, g_hbm, out_hbm,
                    src_v, dst_v, rows_v, z_v, acc_sh, g_sh,
                    gsem_a, gsem_b, ssem_a, ssem_b):
        cid = lax.axis_index("c")
        sid = lax.axis_index("s")
        wid = cid * NS + sid
        pltpu.sync_copy(src_hbm.at[wid], src_v)
        pltpu.sync_copy(dst_hbm.at[wid], dst_v)
        gbase = jnp.minimum(sid * rpt_g, n_g - rpt_g)
        pltpu.sync_copy(g_hbm.at[pl.ds(gbase, rpt_g)],
                        g_sh.at[pl.ds(gbase, rpt_g)])

        def zero_body(i, c):
            z_v[i] = jnp.zeros((feat,), jnp.float32)
            return c

        lax.fori_loop(0, rpt, zero_body, 0)
        pltpu.sync_copy(z_v, acc_sh.at[pl.ds(sid * rpt, rpt)])
        plsc.subcore_barrier()

        sems = {"A": (gsem_a, ssem_a), "B": (gsem_b, ssem_b)}

        def issue_group(grp, base):
            gsem, _ = sems[grp]
            off = 0 if grp == "A" else 4
            for b in range(4):
                pltpu.async_copy(g_sh.at[src_v.at[base + b]],
                                 rows_v.at[off + b], gsem)

        def process_group(grp, j, nxt):
            gsem, ssem = sems[grp]
            off = 0 if grp == "A" else 4
            for b in range(4):
                pltpu.make_async_copy(g_sh.at[src_v.at[j + b]],
                                      rows_v.at[off + b], gsem).wait()
            for b in range(4):
                pltpu.async_copy(rows_v.at[off + b],
                                 acc_sh.at[dst_v.at[j + b]], ssem, add=True)
            for b in range(4):
                pltpu.make_async_copy(rows_v.at[off + b],
                                      acc_sh.at[dst_v.at[j + b]], ssem).wait()
            for b in range(4):
                pltpu.async_copy(g_sh.at[src_v.at[jnp.minimum(nxt + b,
                                                              nch - 1)]],
                                 rows_v.at[off + b], gsem)

        issue_group("A", 0)
        issue_group("B", 4)

        def body(i, c):
            j = 8 * i
            process_group("A", j, j + 8)
            process_group("B", j + 4, j + 12)
            return c

        lax.fori_loop(0, nch // 8, body, 0)
        for b in range(4):
            pltpu.make_async_copy(g_sh.at[src_v.at[nch - 1]],
                                  rows_v.at[b], gsem_a).wait()
        for b in range(4):
            pltpu.make_async_copy(g_sh.at[src_v.at[nch - 1]],
                                  rows_v.at[4 + b], gsem_b).wait()
        plsc.subcore_barrier()
        pltpu.sync_copy(acc_sh.at[pl.ds(sid * rpt, rpt)],
                        out_hbm.at[cid, pl.ds(sid * rpt, rpt)])

    return edge_kernel


def _tc1_body(dp_ref, x_ref, w_ref, g_ref):
    deg = dp_ref[0] + dp_ref[1] + 1.0
    dinv = lax.rsqrt(deg)
    h = jnp.dot(x_ref[...], w_ref[...], preferred_element_type=jnp.float32)
    g_ref[...] = h * dinv


def _tc2_body(sp_ref, dp_ref, g1_ref, b1_ref, g2_ref):
    deg = dp_ref[0] + dp_ref[1] + 1.0
    dinv = lax.rsqrt(deg)
    s = sp_ref[0] + sp_ref[1] + g1_ref[...]
    h1 = jnp.maximum(s * dinv + b1_ref[...], 0.0)
    g2_ref[...] = h1 * dinv


def _tc3_body(sp_ref, dp_ref, g2_ref, w2_ref, b2_ref, o_ref):
    deg = dp_ref[0] + dp_ref[1] + 1.0
    dinv = lax.rsqrt(deg)
    s = sp_ref[0] + sp_ref[1] + g2_ref[...]
    t = jnp.dot(s, w2_ref[...], preferred_element_type=jnp.float32)
    o_ref[...] = jax.nn.sigmoid(t * dinv + b2_ref[...])


def kernel(x, edge_index, batch, W1, b1, W2, b2):
    n, f = x.shape
    h = W1.shape[1]
    e = edge_index.shape[1]
    npad = -(-(n + 1) // 256) * 256
    ew = -(-e // NW)
    nch = -(-ew // CH)
    nch = -(-nch // 8) * 8
    tot = NW * nch * CH

    src = edge_index[0]
    dst = edge_index[1]
    junk = n + (jnp.arange(tot - e, dtype=jnp.int32) % (npad - n))
    src3 = jnp.concatenate(
        [src, jnp.zeros((tot - e,), jnp.int32)]).reshape(NW, nch, CH)
    dst3 = jnp.concatenate([dst, junk]).reshape(NW, nch, CH)

    deg_k = _make_deg_kernel(nch, npad)
    edge_k = _make_edge_kernel(nch, npad, h, n)

    degp = deg_k(dst3)
    dp = degp.reshape(NC, npad, 1)

    blk = 2000
    grid = (n // blk,)
    dp_spec = pl.BlockSpec((NC, blk, 1), lambda i: (0, i, 0))
    sp_spec = pl.BlockSpec((NC, blk, h), lambda i: (0, i, 0))
    g_spec = pl.BlockSpec((blk, h), lambda i: (i, 0))

    g1 = pl.pallas_call(
        _tc1_body,
        grid=grid,
        in_specs=[dp_spec,
                  pl.BlockSpec((blk, f), lambda i: (i, 0)),
                  pl.BlockSpec((f, h), lambda i: (0, 0))],
        out_specs=g_spec,
        out_shape=jax.ShapeDtypeStruct((n, h), jnp.float32),
    )(dp, x, W1)

    s1p = edge_k(src3, dst3, g1)

    g2 = pl.pallas_call(
        _tc2_body,
        grid=grid,
        in_specs=[sp_spec, dp_spec, g_spec,
                  pl.BlockSpec((1, h), lambda i: (0, 0))],
        out_specs=g_spec,
        out_shape=jax.ShapeDtypeStruct((n, h), jnp.float32),
    )(s1p, dp, g1, b1.reshape(1, h))

    s2p = edge_k(src3, dst3, g2)

    out = pl.pallas_call(
        _tc3_body,
        grid=grid,
        in_specs=[sp_spec, dp_spec, g_spec,
                  pl.BlockSpec((h, 1), lambda i: (0, 0)),
                  pl.BlockSpec((1, 1), lambda i: (0, 0))],
        out_specs=pl.BlockSpec((blk, 1), lambda i: (i, 0)),
        out_shape=jax.ShapeDtypeStruct((n, 1), jnp.float32),
    )(s2p, dp, g2, W2, b2.reshape(1, 1))

    return out
